# baseline (speedup 1.0000x reference)
import functools

import jax
import jax.numpy as jnp
from jax.experimental import pallas as pl
from jax.experimental.pallas import tpu as pltpu

SEQ = 2048
EXPERTS = 16
HIDDEN = 2048
TOPK = 2
CAPACITY = 256
BLK = 256
NBLK = SEQ // BLK


def _inclusive_cumsum_rows(x):
    n = x.shape[0]
    d = 1
    while d < n:
        shifted = jnp.concatenate(
            [jnp.zeros((d, x.shape[1]), x.dtype), x[:-d, :]], axis=0
        )
        x = x + shifted
        d *= 2
    return x


def _routing_body(hs_ref, wg_ref, rp_ref, p0_ref, p1_ref, c0_ref, cnt_ref,
                  laux_ref, rate_ref, offs0, offs1, sumg):
    i = pl.program_id(0)

    @pl.when(i == 0)
    def _init():
        offs0[...] = jnp.zeros_like(offs0)
        offs1[...] = jnp.zeros_like(offs1)
        sumg[...] = jnp.zeros_like(sumg)

    x = hs_ref[...]
    w = wg_ref[...]
    logits = jax.lax.dot_general(
        x, w, (((1,), (1,)), ((), ())), preferred_element_type=jnp.float32
    )

    m = jnp.max(logits, axis=1, keepdims=True)
    ex = jnp.exp(logits - m)
    g = ex / jnp.sum(ex, axis=1, keepdims=True)

    iota = jax.lax.broadcasted_iota(jnp.int32, (BLK, EXPERTS), 1)
    v0 = jnp.max(g, axis=1, keepdims=True)
    idx0 = jnp.min(jnp.where(g == v0, iota, EXPERTS), axis=1, keepdims=True)
    m0 = iota == idx0
    g_ex = jnp.where(m0, -jnp.inf, g)
    v1 = jnp.max(g_ex, axis=1, keepdims=True)
    idx1 = jnp.min(jnp.where(g_ex == v1, iota, EXPERTS), axis=1, keepdims=True)
    m1 = iota == idx1

    gates_s = jnp.maximum(v0 + v1, jnp.finfo(jnp.float32).eps)
    rp_ref[...] = g / gates_s

    m0f = m0.astype(jnp.float32)
    m1f = m1.astype(jnp.float32)
    inc0 = _inclusive_cumsum_rows(m0f)
    inc1 = _inclusive_cumsum_rows(m1f)
    exc0 = inc0 - m0f
    exc1 = inc1 - m1f

    p0_ref[...] = jnp.where(m0, offs0[...] + exc0, -1.0)
    p1_ref[...] = jnp.where(m1, offs1[...] + exc1, -1.0)

    offs0[...] = offs0[...] + inc0[BLK - 1 : BLK, :]
    offs1[...] = offs1[...] + inc1[BLK - 1 : BLK, :]
    sumg[...] = sumg[...] + jnp.sum(g, axis=0, keepdims=True)
    c0_ref[...] = offs0[...]

    @pl.when(i == NBLK - 1)
    def _finish():
        ctot = offs0[...] + offs1[...]
        cnt_ref[...] = ctot.astype(jnp.int32)
        inv_s = 1.0 / SEQ
        laux = (EXPERTS * EXPERTS) * jnp.mean(
            (ctot * inv_s) * (sumg[...] * inv_s)
        )
        laux_ref[0, 0] = laux
        rate_ref[0, 0] = jnp.sum(jnp.minimum(ctot, float(CAPACITY))) / (
            SEQ * TOPK
        )


def _materialize_body(rp_ref, p0_ref, p1_ref, c0_ref, comb_ref, disp_ref):
    rp = rp_ref[...]
    p0 = p0_ref[...]
    p1p = p1_ref[...]
    c0 = c0_ref[...]

    p1 = jnp.where(p1p >= 0.0, p1p + c0, -1.0)
    tp = jnp.maximum(p0, p1)
    valid = jnp.logical_and(tp >= 0.0, tp < float(CAPACITY))
    tpc = jnp.where(valid, tp, -1.0).astype(jnp.int32)

    cap_iota = jax.lax.broadcasted_iota(
        jnp.int32, (BLK, EXPERTS, CAPACITY), 2
    )
    disp = tpc[:, :, None] == cap_iota
    disp_ref[...] = disp
    comb_ref[...] = jnp.where(disp, rp[:, :, None], 0.0)


@functools.partial(jax.jit)
def _run(hs, wg):
    meta_spec = pl.BlockSpec((BLK, EXPERTS), lambda i: (i, 0))
    vec_spec = pl.BlockSpec((1, EXPERTS), lambda i: (0, 0))
    smem_spec = pl.BlockSpec(memory_space=pltpu.SMEM)

    rp, p0, p1, c0, cnt, laux, rate = pl.pallas_call(
        _routing_body,
        grid=(NBLK,),
        in_specs=[
            pl.BlockSpec((BLK, HIDDEN), lambda i: (i, 0)),
            pl.BlockSpec((EXPERTS, HIDDEN), lambda i: (0, 0)),
        ],
        out_specs=[meta_spec, meta_spec, meta_spec, vec_spec, vec_spec,
                   smem_spec, smem_spec],
        out_shape=[
            jax.ShapeDtypeStruct((SEQ, EXPERTS), jnp.float32),
            jax.ShapeDtypeStruct((SEQ, EXPERTS), jnp.float32),
            jax.ShapeDtypeStruct((SEQ, EXPERTS), jnp.float32),
            jax.ShapeDtypeStruct((1, EXPERTS), jnp.float32),
            jax.ShapeDtypeStruct((1, EXPERTS), jnp.int32),
            jax.ShapeDtypeStruct((1, 1), jnp.float32),
            jax.ShapeDtypeStruct((1, 1), jnp.float32),
        ],
        scratch_shapes=[
            pltpu.VMEM((1, EXPERTS), jnp.float32),
            pltpu.VMEM((1, EXPERTS), jnp.float32),
            pltpu.VMEM((1, EXPERTS), jnp.float32),
        ],
    )(hs, wg)

    comb, disp = pl.pallas_call(
        _materialize_body,
        grid=(NBLK,),
        in_specs=[meta_spec, meta_spec, meta_spec, vec_spec],
        out_specs=[
            pl.BlockSpec((BLK, EXPERTS, CAPACITY), lambda i: (i, 0, 0)),
            pl.BlockSpec((BLK, EXPERTS, CAPACITY), lambda i: (i, 0, 0)),
        ],
        out_shape=[
            jax.ShapeDtypeStruct((SEQ, EXPERTS, CAPACITY), jnp.float32),
            jax.ShapeDtypeStruct((SEQ, EXPERTS, CAPACITY), jnp.bool_),
        ],
    )(rp, p0, p1, c0)

    return (
        laux.reshape(()),
        rate.reshape(()),
        comb,
        disp,
        cnt.reshape(EXPERTS),
    )


def kernel(hidden_states, wg_weight):
    hs = hidden_states.reshape(-1, HIDDEN).astype(jnp.float32)
    return _run(hs, wg_weight)
